# Initial kernel scaffold; baseline (speedup 1.0000x reference)
import functools

import jax
import jax.numpy as jnp
from jax import lax
from jax.experimental import pallas as pl
from jax.experimental.pallas import tpu as pltpu
from jax.experimental.pallas import tpu_sc as plsc

_N = 10000
_E = 320000
_H = 3
_C = 32
_DH = _H * _C
_DIN = 128
_DOUT = 64

_NC = 2
_NS = 16
_NW = _NC * _NS
_EPW = _E // _NW
_K = 80
_NCHUNK = _EPW // _K
_RPT = _N // _NS

_SW = 16


def _sc_edge_body(edge_hbm, xl_hbm, alt_hbm, z96_hbm, z16_hbm,
                  acc_hbm, s_hbm,
                  tab, src_idx, dst_idx, rows, exrow, exbuf,
                  acc_sh, s_sh, sem):
    cid = lax.axis_index("c")
    sid = lax.axis_index("s")
    wid = cid * _NS + sid

    r0 = sid * _RPT
    pltpu.sync_copy(z96_hbm.at[pl.ds(r0, _RPT)], acc_sh.at[pl.ds(r0, _RPT)])
    pltpu.sync_copy(z16_hbm.at[pl.ds(r0, _RPT)], s_sh.at[pl.ds(r0, _RPT)])

    for h in range(2 * _H):
        pltpu.sync_copy(alt_hbm.at[h], tab.at[pl.ds(h * _N, _N)])

    def _zr(k, carry):
        exrow[k] = jnp.zeros((16,), jnp.float32)
        return carry
    lax.fori_loop(0, _K, _zr, 0)

    plsc.subcore_barrier()

    lane = lax.iota(jnp.int32, 16)
    ebase = wid * _EPW

    def _chunk(ci, carry):
        base = ebase + ci * _K
        pltpu.sync_copy(edge_hbm.at[0, pl.ds(base, _K)], src_idx)
        pltpu.sync_copy(edge_hbm.at[1, pl.ds(base, _K)], dst_idx)
        pltpu.async_copy(xl_hbm.at[src_idx], rows, sem).wait()

        for kk in range(0, _K, 16):
            sv = src_idx[pl.ds(kk, 16)]
            dv = dst_idx[pl.ds(kk, 16)]
            for h in range(_H):
                als = plsc.load_gather(tab, [sv + h * _N])
                ald = plsc.load_gather(tab, [dv + (_H + h) * _N])
                e = als + ald
                e = jnp.where(e > 0.0, e, 0.2 * e)
                ex = jnp.exp(e)
                exbuf[pl.ds(h * _K + kk, 16)] = ex
                plsc.store_scatter(
                    exrow,
                    [lane + kk, jnp.full((16,), h, jnp.int32)],
                    ex)

        def _edge(k, carry):
            for h in range(_H):
                b = plsc.load_gather(
                    exbuf, [jnp.full((16,), h * _K, jnp.int32) + k])
                c0 = h * _C
                rows[k, pl.ds(c0, 16)] = rows[k, pl.ds(c0, 16)] * b
                rows[k, pl.ds(c0 + 16, 16)] = rows[k, pl.ds(c0 + 16, 16)] * b
            return carry
        lax.fori_loop(0, _K, _edge, 0)

        pltpu.sync_copy(rows, acc_sh.at[dst_idx], add=True)
        pltpu.sync_copy(exrow, s_sh.at[dst_idx], add=True)
        return carry

    lax.fori_loop(0, _NCHUNK, _chunk, 0)

    plsc.subcore_barrier()

    ro = cid * _N + r0
    pltpu.sync_copy(acc_sh.at[pl.ds(r0, _RPT)], acc_hbm.at[pl.ds(ro, _RPT)])
    pltpu.sync_copy(s_sh.at[pl.ds(r0, _RPT)], s_hbm.at[pl.ds(ro, _RPT)])


_sc_edge_pass = functools.partial(
    pl.kernel,
    out_type=[
        jax.ShapeDtypeStruct((2 * _N, _DH), jnp.float32),
        jax.ShapeDtypeStruct((2 * _N, _SW), jnp.float32),
    ],
    mesh=plsc.VectorSubcoreMesh(core_axis_name="c", subcore_axis_name="s"),
    scratch_types=[
        pltpu.VMEM((2 * _H * _N,), jnp.float32),
        pltpu.VMEM((_K,), jnp.int32),
        pltpu.VMEM((_K,), jnp.int32),
        pltpu.VMEM((_K, _DH), jnp.float32),
        pltpu.VMEM((_K, _SW), jnp.float32),
        pltpu.VMEM((_H * _K,), jnp.float32),
        pltpu.VMEM_SHARED((_N, _DH), jnp.float32),
        pltpu.VMEM_SHARED((_N, _SW), jnp.float32),
        pltpu.SemaphoreType.DMA,
    ],
)(_sc_edge_body)


def _head_expander():
    hi = lax.broadcasted_iota(jnp.int32, (_H, _DH), 0)
    cj = lax.broadcasted_iota(jnp.int32, (_H, _DH), 1)
    return jnp.where(cj // _C == hi, 1.0, 0.0).astype(jnp.float32)


def _tc_pre_body(x_ref, w_ref, a6_ref, xl_ref, al_ref):
    xl = jnp.dot(x_ref[...], w_ref[...], preferred_element_type=jnp.float32)
    xl_ref[...] = xl
    al_ref[...] = jnp.dot(xl, a6_ref[...], preferred_element_type=jnp.float32)


def _normalize_bn(acc_ref, s_ref, b, g, be):
    acc = acc_ref[0:_N, :] + acc_ref[_N:2 * _N, :]
    s3 = s_ref[0:_N, 0:_H] + s_ref[_N:2 * _N, 0:_H]
    rec = 1.0 / (s3 + 1e-16)
    rec96 = jnp.dot(rec, _head_expander(), preferred_element_type=jnp.float32)
    h = acc * rec96 + b
    h = jnp.maximum(h, 0.0)
    mu = jnp.mean(h, axis=0)
    var = jnp.mean((h - mu) ** 2, axis=0)
    return (h - mu) * lax.rsqrt(var + 1e-5) * g + be


def _tc_mid_body(acc_ref, s_ref, b_ref, g_ref, be_ref, w_ref, a6_ref,
                 h1_ref, xl2_ref, al2_ref):
    h1 = _normalize_bn(acc_ref, s_ref, b_ref[...], g_ref[...], be_ref[...])
    h1_ref[...] = h1
    xl2 = jnp.dot(h1, w_ref[...], preferred_element_type=jnp.float32)
    xl2_ref[...] = xl2
    al2_ref[...] = jnp.dot(xl2, a6_ref[...], preferred_element_type=jnp.float32)


def _tc_fin_body(acc_ref, s_ref, b_ref, g_ref, be_ref, h1_ref, wf_ref, bf_ref,
                 out_ref):
    h2 = _normalize_bn(acc_ref, s_ref, b_ref[...], g_ref[...], be_ref[...])
    hs = h1_ref[...] + h2
    out_ref[...] = (
        jnp.dot(hs, wf_ref[...], preferred_element_type=jnp.float32)
        + bf_ref[...])


_tc_pre = pl.pallas_call(
    _tc_pre_body,
    out_shape=[
        jax.ShapeDtypeStruct((_N, _DH), jnp.float32),
        jax.ShapeDtypeStruct((_N, 2 * _H), jnp.float32),
    ],
)

_tc_mid = pl.pallas_call(
    _tc_mid_body,
    out_shape=[
        jax.ShapeDtypeStruct((_N, _DH), jnp.float32),
        jax.ShapeDtypeStruct((_N, _DH), jnp.float32),
        jax.ShapeDtypeStruct((_N, 2 * _H), jnp.float32),
    ],
)

_tc_fin = pl.pallas_call(
    _tc_fin_body,
    out_shape=jax.ShapeDtypeStruct((_N, _DOUT), jnp.float32),
)


def _attn_matrix(a_src, a_dst):
    eye = jnp.eye(_H, dtype=jnp.float32)
    asrc = (a_src.reshape(_H, _C)[:, :, None] * eye[:, None, :]).reshape(_DH, _H)
    adst = (a_dst.reshape(_H, _C)[:, :, None] * eye[:, None, :]).reshape(_DH, _H)
    return jnp.concatenate([asrc, adst], axis=1)


def kernel(x, edge_index, W1, a_src1, a_dst1, b1, g1, be1,
           W2, a_src2, a_dst2, b2, g2, be2, Wf, bf):
    z96 = jnp.zeros((_N, _DH), jnp.float32)
    z16 = jnp.zeros((_N, _SW), jnp.float32)

    a61 = _attn_matrix(a_src1, a_dst1)
    a62 = _attn_matrix(a_src2, a_dst2)

    xl1, al1 = _tc_pre(x, W1, a61)
    acc1, s1 = _sc_edge_pass(edge_index, xl1, al1.T, z96, z16)
    h1, xl2, al2 = _tc_mid(acc1, s1, b1, g1, be1, W2, a62)
    acc2, s2 = _sc_edge_pass(edge_index, xl2, al2.T, z96, z16)
    out = _tc_fin(acc2, s2, b2, g2, be2, h1, Wf, bf)
    return out


# speedup vs baseline: 31.4760x; 31.4760x over previous
import functools

import jax
import jax.numpy as jnp
from jax import lax
from jax.experimental import pallas as pl
from jax.experimental.pallas import tpu as pltpu
from jax.experimental.pallas import tpu_sc as plsc

_N = 10000
_E = 320000
_H = 3
_C = 32
_DH = _H * _C
_DIN = 128
_DOUT = 64

_NP = 10240
_F = 128
_FH = _F // 2

_NC = 2
_NS = 16
_K = 128
_TCH = _E // _K
_CPS = -(-_TCH // _NS)
_RPT = _NP // _NS


def _sc_edge_body(edge_hbm, xl_hbm, alt_hbm, acc_hbm,
                  tab, src_idx, src2_idx, dst_idx, rows, exbuf,
                  acc_sh, sem):
    cid = lax.axis_index("c")
    sid = lax.axis_index("s")

    def _zr(k, carry):
        for c in range(_FH // 16):
            rows[k, pl.ds(c * 16, 16)] = jnp.zeros((16,), jnp.float32)
        return carry
    lax.fori_loop(0, _K, _zr, 0)
    r0 = sid * _RPT
    for t in range(_RPT // _K):
        pltpu.sync_copy(rows, acc_sh.at[pl.ds(r0 + t * _K, _K)])

    def _ze(i, carry):
        exbuf[pl.ds(_H * _K + i * 16, 16)] = jnp.zeros((16,), jnp.float32)
        return carry
    lax.fori_loop(0, (16 - _H) * _K // 16, _ze, 0)

    for h in range(2 * _H):
        pltpu.sync_copy(alt_hbm.at[pl.ds(h * _NP, _NP)],
                        tab.at[pl.ds(h * _NP, _NP)])

    plsc.subcore_barrier()

    lane = lax.iota(jnp.int32, 16)

    def _chunk(ci, carry):
        chunk = ci * _NS + sid

        @pl.when(chunk < _TCH)
        def _():
            base = chunk * _K
            pltpu.sync_copy(edge_hbm.at[pl.ds(base, _K)], src_idx)
            pltpu.sync_copy(edge_hbm.at[pl.ds(_E + base, _K)], dst_idx)
            for kk in range(0, _K, 16):
                sv = src_idx[pl.ds(kk, 16)]
                src2_idx[pl.ds(kk, 16)] = sv + sv + cid
            pltpu.async_copy(xl_hbm.at[src2_idx], rows, sem).wait()

            for kk in range(0, _K, 16):
                sv = src_idx[pl.ds(kk, 16)]
                dv = dst_idx[pl.ds(kk, 16)]
                for h in range(_H):
                    als = plsc.load_gather(tab, [sv + h * _NP])
                    ald = plsc.load_gather(tab, [dv + (_H + h) * _NP])
                    e = als + ald
                    e = jnp.where(e > 0.0, e, 0.2 * e)
                    exbuf[pl.ds(h * _K + kk, 16)] = jnp.exp(e)

            @pl.when(cid == 0)
            def _():
                def _edge0(k, carry2):
                    for h in range(2):
                        b = plsc.load_gather(
                            exbuf, [jnp.full((16,), h * _K, jnp.int32) + k])
                        c0 = h * _C
                        rows[k, pl.ds(c0, 16)] = rows[k, pl.ds(c0, 16)] * b
                        rows[k, pl.ds(c0 + 16, 16)] = (
                            rows[k, pl.ds(c0 + 16, 16)] * b)
                    return carry2
                lax.fori_loop(0, _K, _edge0, 0)

            @pl.when(cid == 1)
            def _():
                def _edge1(k, carry2):
                    b = plsc.load_gather(
                        exbuf, [jnp.full((16,), 2 * _K, jnp.int32) + k])
                    rows[k, pl.ds(0, 16)] = rows[k, pl.ds(0, 16)] * b
                    rows[k, pl.ds(16, 16)] = rows[k, pl.ds(16, 16)] * b
                    bt = plsc.load_gather(exbuf, [lane * _K + k])
                    rows[k, pl.ds(32, 16)] = rows[k, pl.ds(32, 16)] * bt
                    return carry2
                lax.fori_loop(0, _K, _edge1, 0)

            pltpu.sync_copy(rows, acc_sh.at[dst_idx], add=True)
        return carry

    lax.fori_loop(0, _CPS, _chunk, 0)

    plsc.subcore_barrier()

    ro = cid * _NP + r0
    pltpu.sync_copy(acc_sh.at[pl.ds(r0, _RPT)], acc_hbm.at[pl.ds(ro, _RPT)])


@functools.cache
def _sc_edge_pass():
    return pl.kernel(
        _sc_edge_body,
        out_type=jax.ShapeDtypeStruct((2 * _NP, _FH), jnp.float32),
        mesh=plsc.VectorSubcoreMesh(core_axis_name="c", subcore_axis_name="s",
                                    num_cores=_NC, num_subcores=_NS),
        compiler_params=pltpu.CompilerParams(use_tc_tiling_on_sc=False,
                                             needs_layout_passes=False),
        scratch_types=[
            pltpu.VMEM((2 * _H * _NP,), jnp.float32),
            pltpu.VMEM((_K,), jnp.int32),
            pltpu.VMEM((_K,), jnp.int32),
            pltpu.VMEM((_K,), jnp.int32),
            pltpu.VMEM((_K, _FH), jnp.float32),
            pltpu.VMEM((16 * _K,), jnp.float32),
            pltpu.VMEM_SHARED((_NP, _FH), jnp.float32),
            pltpu.SemaphoreType.DMA,
        ],
    )


def _ones_cols():
    j = lax.broadcasted_iota(jnp.int32, (1, _F), 1)
    return jnp.where((j >= _DH) & (j < _DH + _H), 1.0, 0.0).astype(jnp.float32)


def _head_expander():
    hi = lax.broadcasted_iota(jnp.int32, (_H, _DH), 0)
    cj = lax.broadcasted_iota(jnp.int32, (_H, _DH), 1)
    return jnp.where(cj // _C == hi, 1.0, 0.0).astype(jnp.float32)


def _tc_pre_body(x_ref, w_ref, a6_ref, xl_ref, al_ref):
    xl = jnp.dot(x_ref[...], w_ref[...], preferred_element_type=jnp.float32)
    xl = xl + _ones_cols()
    xl_ref[...] = xl
    al_ref[...] = jnp.dot(xl, a6_ref[...], preferred_element_type=jnp.float32)


def _normalize_bn(acc_ref, b, g, be):
    acc = jnp.concatenate(
        [acc_ref[0:_N, :], acc_ref[_NP:_NP + _N, 0:_C]], axis=1)
    s3 = acc_ref[_NP:_NP + _N, _C:_C + _H]
    rec = 1.0 / (s3 + 1e-16)
    rec96 = jnp.dot(rec, _head_expander(), preferred_element_type=jnp.float32)
    h = acc * rec96 + b
    h = jnp.maximum(h, 0.0)
    mu = jnp.mean(h, axis=0)
    var = jnp.mean((h - mu) ** 2, axis=0)
    return (h - mu) * lax.rsqrt(var + 1e-5) * g + be


def _tc_mid_body(acc_ref, b_ref, g_ref, be_ref, w_ref, a6_ref,
                 h1_ref, xl2_ref, al2_ref):
    h1 = _normalize_bn(acc_ref, b_ref[...], g_ref[...], be_ref[...])
    h1_ref[...] = h1
    xl2 = jnp.dot(h1, w_ref[...], preferred_element_type=jnp.float32)
    xl2 = xl2 + _ones_cols()
    xl2_ref[...] = xl2
    al2_ref[...] = jnp.dot(xl2, a6_ref[...], preferred_element_type=jnp.float32)


def _tc_fin_body(acc_ref, b_ref, g_ref, be_ref, h1_ref, wf_ref, bf_ref,
                 out_ref):
    h2 = _normalize_bn(acc_ref, b_ref[...], g_ref[...], be_ref[...])
    hs = h1_ref[...] + h2
    out_ref[...] = (
        jnp.dot(hs, wf_ref[...], preferred_element_type=jnp.float32)
        + bf_ref[...])


_tc_pre = pl.pallas_call(
    _tc_pre_body,
    out_shape=[
        jax.ShapeDtypeStruct((_N, _F), jnp.float32),
        jax.ShapeDtypeStruct((_N, 2 * _H), jnp.float32),
    ],
)

_tc_mid = pl.pallas_call(
    _tc_mid_body,
    out_shape=[
        jax.ShapeDtypeStruct((_N, _DH), jnp.float32),
        jax.ShapeDtypeStruct((_N, _F), jnp.float32),
        jax.ShapeDtypeStruct((_N, 2 * _H), jnp.float32),
    ],
)

_tc_fin = pl.pallas_call(
    _tc_fin_body,
    out_shape=jax.ShapeDtypeStruct((_N, _DOUT), jnp.float32),
)


def _attn_matrix(a_src, a_dst):
    eye = jnp.eye(_H, dtype=jnp.float32)
    asrc = (a_src.reshape(_H, _C)[:, :, None] * eye[:, None, :]).reshape(_DH, _H)
    adst = (a_dst.reshape(_H, _C)[:, :, None] * eye[:, None, :]).reshape(_DH, _H)
    a6 = jnp.concatenate([asrc, adst], axis=1)
    return jnp.pad(a6, ((0, _F - _DH), (0, 0)))


def _flat_tables(al):
    return jnp.pad(al.T, ((0, 0), (0, _NP - _N))).reshape(-1)


def kernel(x, edge_index, W1, a_src1, a_dst1, b1, g1, be1,
           W2, a_src2, a_dst2, b2, g2, be2, Wf, bf):
    edges = edge_index.reshape(-1)
    w1p = jnp.pad(W1, ((0, 0), (0, _F - _DH)))
    w2p = jnp.pad(W2, ((0, 0), (0, _F - _DH)))
    a61 = _attn_matrix(a_src1, a_dst1)
    a62 = _attn_matrix(a_src2, a_dst2)

    sc_pass = _sc_edge_pass()
    xl1, al1 = _tc_pre(x, w1p, a61)
    acc1 = sc_pass(edges, xl1.reshape(2 * _N, _FH), _flat_tables(al1))
    h1, xl2, al2 = _tc_mid(acc1, b1, g1, be1, w2p, a62)
    acc2 = sc_pass(edges, xl2.reshape(2 * _N, _FH), _flat_tables(al2))
    out = _tc_fin(acc2, b2, g2, be2, h1, Wf, bf)
    return out


# speedup vs baseline: 47.4284x; 1.5068x over previous
import functools

import jax
import jax.numpy as jnp
from jax import lax
from jax.experimental import pallas as pl
from jax.experimental.pallas import tpu as pltpu
from jax.experimental.pallas import tpu_sc as plsc

_N = 10000
_E = 320000
_H = 3
_C = 32
_DH = _H * _C
_DIN = 128
_DOUT = 64

_NP = 10240
_F = 128
_FH = _F // 2

_NC = 2
_NS = 16
_K = 128
_TCH = _E // _K
_CPS = -(-_TCH // _NS)
_RPT = _NP // _NS


_NSLOT = _CPS


def _sc_edge_body(edge_hbm, xl_hbm, alt_hbm, acc_hbm,
                  tab, ib0, ib1, ib2, sb0, sb1, sb2, db0, db1, db2,
                  rw0, rw1, rw2, exbuf, acc_sh,
                  is0, is1, is2, gs0, gs1, gs2, ss0, ss1, ss2):
    cid = lax.axis_index("c")
    sid = lax.axis_index("s")
    idxb = [ib0, ib1, ib2]
    srcb = [sb0, sb1, sb2]
    dstb = [db0, db1, db2]
    rows = [rw0, rw1, rw2]
    isem = [is0, is1, is2]
    gsem = [gs0, gs1, gs2]
    ssem = [ss0, ss1, ss2]

    def _zr(k, carry):
        for c in range(_FH // 16):
            rw0[k, pl.ds(c * 16, 16)] = jnp.zeros((16,), jnp.float32)
        return carry
    lax.fori_loop(0, _K, _zr, 0)
    r0 = sid * _RPT
    for t in range(_RPT // _K):
        pltpu.sync_copy(rw0, acc_sh.at[pl.ds(r0 + t * _K, _K)])

    def _ze(i, carry):
        exbuf[pl.ds(_H * _K + i * 16, 16)] = jnp.zeros((16,), jnp.float32)
        return carry
    lax.fori_loop(0, (16 - _H) * _K // 16, _ze, 0)

    for h in range(2 * _H):
        pltpu.sync_copy(alt_hbm.at[pl.ds(h * _NP, _NP)],
                        tab.at[pl.ds(h * _NP, _NP)])

    plsc.subcore_barrier()

    lane = lax.iota(jnp.int32, 16)

    def _chbase(i):
        return jnp.minimum(i * _NS + sid, _TCH - 1) * (2 * _K)

    def _idx_issue(j, bj):
        pltpu.async_copy(edge_hbm.at[pl.ds(_chbase(j), 2 * _K)],
                         idxb[bj], isem[bj])

    def _gather_issue(j, bj):
        pltpu.make_async_copy(edge_hbm.at[pl.ds(_chbase(j), 2 * _K)],
                              idxb[bj], isem[bj]).wait()
        valid = (j * _NS + sid) < _TCH
        for kk in range(0, _K, 16):
            sv = idxb[bj][pl.ds(kk, 16)]
            srcb[bj][pl.ds(kk, 16)] = sv + sv + cid
            dv = idxb[bj][pl.ds(_K + kk, 16)]
            dstb[bj][pl.ds(kk, 16)] = jnp.where(
                valid, dv, jnp.full((16,), _N, jnp.int32))
        pltpu.async_copy(xl_hbm.at[srcb[bj]], rows[bj], gsem[bj])

    def _compute(b):
        pltpu.make_async_copy(xl_hbm.at[srcb[b]], rows[b], gsem[b]).wait()
        for kk in range(0, _K, 16):
            sv = idxb[b][pl.ds(kk, 16)]
            dv = dstb[b][pl.ds(kk, 16)]
            for h in range(_H):
                als = plsc.load_gather(tab, [sv + h * _NP])
                ald = plsc.load_gather(tab, [dv + (_H + h) * _NP])
                e = als + ald
                e = jnp.where(e > 0.0, e, 0.2 * e)
                exbuf[pl.ds(h * _K + kk, 16)] = jnp.exp(e)

        rw = rows[b]

        @pl.when(cid == 0)
        def _():
            def _edge0(it, carry2):
                for j in range(4):
                    k = it * 4 + j
                    for h in range(2):
                        w = plsc.load_gather(
                            exbuf, [jnp.full((16,), h * _K, jnp.int32) + k])
                        c0 = h * _C
                        rw[k, pl.ds(c0, 16)] = rw[k, pl.ds(c0, 16)] * w
                        rw[k, pl.ds(c0 + 16, 16)] = (
                            rw[k, pl.ds(c0 + 16, 16)] * w)
                return carry2
            lax.fori_loop(0, _K // 4, _edge0, 0)

        @pl.when(cid == 1)
        def _():
            def _edge1(it, carry2):
                for j in range(4):
                    k = it * 4 + j
                    w = plsc.load_gather(
                        exbuf, [jnp.full((16,), 2 * _K, jnp.int32) + k])
                    rw[k, pl.ds(0, 16)] = rw[k, pl.ds(0, 16)] * w
                    rw[k, pl.ds(16, 16)] = rw[k, pl.ds(16, 16)] * w
                    wt = plsc.load_gather(exbuf, [lane * _K + k])
                    rw[k, pl.ds(32, 16)] = rw[k, pl.ds(32, 16)] * wt
                return carry2
            lax.fori_loop(0, _K // 4, _edge1, 0)

        pltpu.async_copy(rows[b], acc_sh.at[dstb[b]], ssem[b], add=True)

    def _scatter_wait(b):
        pltpu.make_async_copy(rows[b], acc_sh.at[dstb[b]], ssem[b]).wait()

    def _slot(i, b, first, prefetch_guard):
        if not first:
            _scatter_wait((b + 2) % 3)
        if prefetch_guard is None:
            _idx_issue(i + 2, (b + 2) % 3)
        elif prefetch_guard:
            @pl.when(i <= _NSLOT - 3)
            def _():
                _idx_issue(i + 2, (b + 2) % 3)
        _gather_issue(i + 1, (b + 1) % 3)
        _compute(b)

    _idx_issue(0, 0)
    _idx_issue(1, 1)
    _gather_issue(0, 0)
    _slot(0, 0, True, None)
    _slot(1, 1, False, None)
    _slot(2, 2, False, None)

    def _group(g, carry):
        i = 3 + g * 3
        _slot(i, 0, False, True)
        _slot(i + 1, 1, False, True)
        _slot(i + 2, 2, False, True)
        return carry
    lax.fori_loop(0, (_NSLOT - 4) // 3, _group, 0)

    _scatter_wait(2)
    _compute(0)
    _scatter_wait(0)

    plsc.subcore_barrier()

    ro = cid * _NP + r0
    pltpu.sync_copy(acc_sh.at[pl.ds(r0, _RPT)], acc_hbm.at[pl.ds(ro, _RPT)])


@functools.cache
def _sc_edge_pass():
    return pl.kernel(
        _sc_edge_body,
        out_type=jax.ShapeDtypeStruct((2 * _NP, _FH), jnp.float32),
        mesh=plsc.VectorSubcoreMesh(core_axis_name="c", subcore_axis_name="s",
                                    num_cores=_NC, num_subcores=_NS),
        compiler_params=pltpu.CompilerParams(use_tc_tiling_on_sc=False,
                                             needs_layout_passes=False),
        scratch_types=(
            [pltpu.VMEM((2 * _H * _NP,), jnp.float32)]
            + [pltpu.VMEM((2 * _K,), jnp.int32)] * 3
            + [pltpu.VMEM((_K,), jnp.int32)] * 3
            + [pltpu.VMEM((_K,), jnp.int32)] * 3
            + [pltpu.VMEM((_K, _FH), jnp.float32)] * 3
            + [pltpu.VMEM((16 * _K,), jnp.float32)]
            + [pltpu.VMEM_SHARED((_NP, _FH), jnp.float32)]
            + [pltpu.SemaphoreType.DMA] * 9
        ),
    )


def _ones_cols():
    j = lax.broadcasted_iota(jnp.int32, (1, _F), 1)
    return jnp.where((j >= _DH) & (j < _DH + _H), 1.0, 0.0).astype(jnp.float32)


def _head_expander():
    hi = lax.broadcasted_iota(jnp.int32, (_H, _DH), 0)
    cj = lax.broadcasted_iota(jnp.int32, (_H, _DH), 1)
    return jnp.where(cj // _C == hi, 1.0, 0.0).astype(jnp.float32)


def _tc_pre_body(x_ref, w_ref, a6_ref, xl_ref, al_ref):
    xl = jnp.dot(x_ref[...], w_ref[...], preferred_element_type=jnp.float32)
    xl = xl + _ones_cols()
    xl_ref[...] = xl
    al_ref[...] = jnp.dot(xl, a6_ref[...], preferred_element_type=jnp.float32)


def _normalize_bn(acc_ref, b, g, be):
    acc = jnp.concatenate(
        [acc_ref[0:_N, :], acc_ref[_NP:_NP + _N, 0:_C]], axis=1)
    s3 = acc_ref[_NP:_NP + _N, _C:_C + _H]
    rec = 1.0 / (s3 + 1e-16)
    rec96 = jnp.dot(rec, _head_expander(), preferred_element_type=jnp.float32)
    h = acc * rec96 + b
    h = jnp.maximum(h, 0.0)
    mu = jnp.mean(h, axis=0)
    var = jnp.mean((h - mu) ** 2, axis=0)
    return (h - mu) * lax.rsqrt(var + 1e-5) * g + be


def _tc_mid_body(acc_ref, b_ref, g_ref, be_ref, w_ref, a6_ref,
                 h1_ref, xl2_ref, al2_ref):
    h1 = _normalize_bn(acc_ref, b_ref[...], g_ref[...], be_ref[...])
    h1_ref[...] = h1
    xl2 = jnp.dot(h1, w_ref[...], preferred_element_type=jnp.float32)
    xl2 = xl2 + _ones_cols()
    xl2_ref[...] = xl2
    al2_ref[...] = jnp.dot(xl2, a6_ref[...], preferred_element_type=jnp.float32)


def _tc_fin_body(acc_ref, b_ref, g_ref, be_ref, h1_ref, wf_ref, bf_ref,
                 out_ref):
    h2 = _normalize_bn(acc_ref, b_ref[...], g_ref[...], be_ref[...])
    hs = h1_ref[...] + h2
    out_ref[...] = (
        jnp.dot(hs, wf_ref[...], preferred_element_type=jnp.float32)
        + bf_ref[...])


_tc_pre = pl.pallas_call(
    _tc_pre_body,
    out_shape=[
        jax.ShapeDtypeStruct((_N, _F), jnp.float32),
        jax.ShapeDtypeStruct((_N, 2 * _H), jnp.float32),
    ],
)

_tc_mid = pl.pallas_call(
    _tc_mid_body,
    out_shape=[
        jax.ShapeDtypeStruct((_N, _DH), jnp.float32),
        jax.ShapeDtypeStruct((_N, _F), jnp.float32),
        jax.ShapeDtypeStruct((_N, 2 * _H), jnp.float32),
    ],
)

_tc_fin = pl.pallas_call(
    _tc_fin_body,
    out_shape=jax.ShapeDtypeStruct((_N, _DOUT), jnp.float32),
)


def _attn_matrix(a_src, a_dst):
    eye = jnp.eye(_H, dtype=jnp.float32)
    asrc = (a_src.reshape(_H, _C)[:, :, None] * eye[:, None, :]).reshape(_DH, _H)
    adst = (a_dst.reshape(_H, _C)[:, :, None] * eye[:, None, :]).reshape(_DH, _H)
    a6 = jnp.concatenate([asrc, adst], axis=1)
    return jnp.pad(a6, ((0, _F - _DH), (0, 0)))


def _flat_tables(al):
    return jnp.pad(al.T, ((0, 0), (0, _NP - _N))).reshape(-1)


def kernel(x, edge_index, W1, a_src1, a_dst1, b1, g1, be1,
           W2, a_src2, a_dst2, b2, g2, be2, Wf, bf):
    edges = edge_index.reshape(2, _TCH, _K).transpose(1, 0, 2).reshape(-1)
    w1p = jnp.pad(W1, ((0, 0), (0, _F - _DH)))
    w2p = jnp.pad(W2, ((0, 0), (0, _F - _DH)))
    a61 = _attn_matrix(a_src1, a_dst1)
    a62 = _attn_matrix(a_src2, a_dst2)

    sc_pass = _sc_edge_pass()
    xl1, al1 = _tc_pre(x, w1p, a61)
    acc1 = sc_pass(edges, xl1.reshape(2 * _N, _FH), _flat_tables(al1))
    h1, xl2, al2 = _tc_mid(acc1, b1, g1, be1, w2p, a62)
    acc2 = sc_pass(edges, xl2.reshape(2 * _N, _FH), _flat_tables(al2))
    out = _tc_fin(acc2, b2, g2, be2, h1, Wf, bf)
    return out
